# baseline (speedup 1.0000x reference)
import functools
import math

import numpy as np
import jax
from jax import lax
import jax.numpy as jnp
from jax.experimental import pallas as pl
from jax.experimental.pallas import tpu as pltpu
from jax.experimental.pallas import tpu_sc as plsc

_V = 32000
_N = 2048
_S = float(np.float32(0.1 / (_V - 2)))
_C = 0.9
_CONST_PER_ROW = (_V - 2) * _S * math.log(_S) + _C * math.log(_C)


_BR = 256
_BC = 3200
_NRB = _N // _BR
_NCB = _V // _BC


def _tc_body(t_ref, x_ref, o_ref, acc_ref):
    i = pl.program_id(0)
    j = pl.program_id(1)

    @pl.when((i == 0) & (j == 0))
    def _():
        acc_ref[0] = 0.0

    xb = x_ref[...]
    t = t_ref[0, 0, :]
    col = jax.lax.broadcasted_iota(jnp.int32, (_BR, _BC), 1) + j * _BC
    xz = jnp.where(col == 0, 0.0, xb)
    rowp = jnp.sum(xz, axis=1)
    rowp = jnp.where(t == 0, 0.0, rowp)
    acc_ref[0] += jnp.sum(rowp)

    @pl.when((i == _NRB - 1) & (j == _NCB - 1))
    def _():
        o_ref[0] = acc_ref[0]


def _tc_dense_sum(x, t3):
    return pl.pallas_call(
        _tc_body,
        grid=(_NRB, _NCB),
        in_specs=[
            pl.BlockSpec((1, 1, _BR), lambda i, j: (i, 0, 0)),
            pl.BlockSpec((_BR, _BC), lambda i, j: (i, j)),
        ],
        out_specs=pl.BlockSpec(memory_space=pltpu.SMEM),
        out_shape=jax.ShapeDtypeStruct((1,), jnp.float32),
        scratch_shapes=[pltpu.SMEM((1,), jnp.float32)],
    )(t3, x)


_NC = 2
_NS = 16
_NW = _NC * _NS
_BPW = _N // _NW


def _sc_body(x_hbm, t_hbm, out_hbm, t_v, idx_v, g_v, acc_v, sem):
    wid = lax.axis_index("s") * _NC + lax.axis_index("c")
    base = wid * _BPW
    pltpu.sync_copy(t_hbm.at[pl.ds(base, _BPW)], t_v)
    for c in range(_BPW // 16):
        t16 = t_v[pl.ds(c * 16, 16)]
        rows = lax.iota(jnp.int32, 16) + (base + c * 16)
        idx_v[pl.ds(c * 16, 16)] = rows * _V + t16
    pltpu.async_copy(x_hbm.at[idx_v], g_v, sem).wait()
    acc = jnp.zeros((16,), jnp.float32)
    cnt = jnp.zeros((16,), jnp.float32)
    for c in range(_BPW // 16):
        t16 = t_v[pl.ds(c * 16, 16)]
        g16 = g_v[pl.ds(c * 16, 16)]
        m = t16 != 0
        acc = acc + jnp.where(m, g16, 0.0)
        cnt = cnt + jnp.where(m, 1.0, 0.0)
    acc_v[...] = acc
    pltpu.sync_copy(acc_v, out_hbm.at[wid, 0])
    acc_v[...] = cnt
    pltpu.sync_copy(acc_v, out_hbm.at[wid, 1])


_sc_gather = functools.partial(
    pl.kernel,
    mesh=plsc.VectorSubcoreMesh(core_axis_name="c", subcore_axis_name="s"),
    out_type=jax.ShapeDtypeStruct((_NW, 2, 16), jnp.float32),
    scratch_types=[
        pltpu.VMEM((_BPW,), jnp.int32),
        pltpu.VMEM((_BPW,), jnp.int32),
        pltpu.VMEM((_BPW,), jnp.float32),
        pltpu.VMEM((16,), jnp.float32),
        pltpu.SemaphoreType.DMA,
    ],
)(_sc_body)


def kernel(x, target):
    t32 = target.astype(jnp.int32)
    t3 = t32.reshape(_NRB, 1, _BR)
    dense = _tc_dense_sum(x, t3)[0]
    parts = _sc_gather(x.reshape(_N * _V), t32)
    g = jnp.sum(parts[:, 0, :])
    k = jnp.sum(parts[:, 1, :])
    return (k * _CONST_PER_ROW - (_S * dense + (_C - _S) * g)) / _N


# speedup vs baseline: 2.7781x; 2.7781x over previous
import math

import numpy as np
import jax
import jax.numpy as jnp
from jax.experimental import pallas as pl
from jax.experimental.pallas import tpu as pltpu

_V = 32000
_N = 2048
_S = float(np.float32(0.1 / (_V - 2)))
_C = 0.9
_CONST_PER_ROW = (_V - 2) * _S * math.log(_S) + _C * math.log(_C)

_BR = 64
_BC = 32000
_NRB = _N // _BR
_NCB = _V // _BC


def _body(t_ref, x_ref, o_ref, acc_ref):
    i = pl.program_id(0)
    j = pl.program_id(1)

    @pl.when((i == 0) & (j == 0))
    def _():
        acc_ref[0] = 0.0
        acc_ref[1] = 0.0
        acc_ref[2] = 0.0

    xb = x_ref[...]
    t = t_ref[0, 0, :]
    col = jax.lax.broadcasted_iota(jnp.int32, (_BR, _BC), 1) + j * _BC
    xz = jnp.where(col == 0, 0.0, xb)
    rowp = jnp.sum(xz, axis=1)
    rowp = jnp.where(t == 0, 0.0, rowp)
    acc_ref[0] += jnp.sum(rowp)
    match = (col == t[:, None]) & (t[:, None] != 0)
    acc_ref[1] += jnp.sum(jnp.where(match, xb, 0.0))

    @pl.when(j == 0)
    def _():
        acc_ref[2] += jnp.sum((t != 0).astype(jnp.float32))

    @pl.when((i == _NRB - 1) & (j == _NCB - 1))
    def _():
        o_ref[0, 0] = (acc_ref[2] * _CONST_PER_ROW
                       - (_S * acc_ref[0] + (_C - _S) * acc_ref[1])) / _N


def kernel(x, target):
    t3 = target.astype(jnp.int32).reshape(_NRB, 1, _BR)
    out = pl.pallas_call(
        _body,
        grid=(_NRB, _NCB),
        in_specs=[
            pl.BlockSpec((1, 1, _BR), lambda i, j: (i, 0, 0)),
            pl.BlockSpec((_BR, _BC), lambda i, j: (i, j)),
        ],
        out_specs=pl.BlockSpec(memory_space=pltpu.SMEM),
        out_shape=jax.ShapeDtypeStruct((1, 1), jnp.float32),
        scratch_shapes=[pltpu.SMEM((3,), jnp.float32)],
    )(t3, x)
    return out[0, 0]


# speedup vs baseline: 2.9552x; 1.0637x over previous
import math

import numpy as np
import jax
import jax.numpy as jnp
from jax.experimental import pallas as pl
from jax.experimental.pallas import tpu as pltpu

_V = 32000
_N = 2048
_S = float(np.float32(0.1 / (_V - 2)))
_C = 0.9
_CONST_PER_ROW = (_V - 2) * _S * math.log(_S) + _C * math.log(_C)

_BR = 128
_BC = 32000
_NRB = _N // _BR
_NCB = _V // _BC


def _body(t_ref, x_ref, o_ref, acc_ref):
    i = pl.program_id(0)
    j = pl.program_id(1)

    @pl.when((i == 0) & (j == 0))
    def _():
        acc_ref[0] = 0.0
        acc_ref[1] = 0.0
        acc_ref[2] = 0.0

    xb = x_ref[...]
    t = t_ref[0, 0, :]
    col = jax.lax.broadcasted_iota(jnp.int32, (_BR, _BC), 1) + j * _BC
    xz = jnp.where(col == 0, 0.0, xb)
    rowp = jnp.sum(xz, axis=1)
    rowp = jnp.where(t == 0, 0.0, rowp)
    acc_ref[0] += jnp.sum(rowp)
    match = (col == t[:, None]) & (t[:, None] != 0)
    acc_ref[1] += jnp.sum(jnp.where(match, xb, 0.0))

    @pl.when(j == 0)
    def _():
        acc_ref[2] += jnp.sum((t != 0).astype(jnp.float32))

    @pl.when((i == _NRB - 1) & (j == _NCB - 1))
    def _():
        o_ref[0, 0] = (acc_ref[2] * _CONST_PER_ROW
                       - (_S * acc_ref[0] + (_C - _S) * acc_ref[1])) / _N


def kernel(x, target):
    t3 = target.astype(jnp.int32).reshape(_NRB, 1, _BR)
    out = pl.pallas_call(
        _body,
        grid=(_NRB, _NCB),
        in_specs=[
            pl.BlockSpec((1, 1, _BR), lambda i, j: (i, 0, 0)),
            pl.BlockSpec((_BR, _BC), lambda i, j: (i, j)),
        ],
        out_specs=pl.BlockSpec(memory_space=pltpu.SMEM),
        out_shape=jax.ShapeDtypeStruct((1, 1), jnp.float32),
        scratch_shapes=[pltpu.SMEM((3,), jnp.float32)],
    )(t3, x)
    return out[0, 0]
